# baseline (speedup 1.0000x reference)
import functools

import jax
import jax.numpy as jnp
from jax import lax
from jax.experimental import pallas as pl
from jax.experimental.pallas import tpu as pltpu
from jax.experimental.pallas import tpu_sc as plsc


def _tables_body(src_ref, dst_ref, w_ref, b_ref, out_ref):
    pid = pl.program_id(0)
    x = jnp.where(pid == 0, src_ref[...], dst_ref[...])
    w = w_ref[0]
    y = lax.dot_general(x, w, (((1,), (1,)), ((), ())),
                        preferred_element_type=jnp.float32)
    out_ref[0] = y + b_ref[0]


def _node_tables(src_feat, dst_feat, Wsd, bsd, N, SD, H):
    return pl.pallas_call(
        _tables_body,
        grid=(2,),
        in_specs=[
            pl.BlockSpec((N, SD), lambda i: (0, 0)),
            pl.BlockSpec((N, SD), lambda i: (0, 0)),
            pl.BlockSpec((1, H, SD), lambda i: (i, 0, 0)),
            pl.BlockSpec((1, 1, H), lambda i: (i, 0, 0)),
        ],
        out_specs=pl.BlockSpec((1, N, H), lambda i: (i, 0, 0)),
        out_shape=jax.ShapeDtypeStruct((2, N, H), jnp.float32),
    )(src_feat, dst_feat, Wsd, bsd)


_NC = 2
_NS = 16
_NW = _NC * _NS

_WIN = 128
_IDXB = 1024


def _make_sc_gather(total, H, N):
    per_w = total // _NW
    n_big = per_w // _IDXB
    assert total % _NW == 0 and per_w % _IDXB == 0 and N % (8 * 10) == 0
    half = total // 2
    mesh = plsc.VectorSubcoreMesh(core_axis_name="c", subcore_axis_name="s")

    @functools.partial(
        pl.kernel,
        mesh=mesh,
        out_type=jax.ShapeDtypeStruct((total, H), jnp.float32),
        scratch_types=[
            pltpu.VMEM_SHARED((N, H), jnp.float32),
            pltpu.VMEM((_IDXB,), jnp.int32),
            pltpu.VMEM((_WIN, H), jnp.float32),
            pltpu.VMEM((_WIN, H), jnp.float32),
            pltpu.SemaphoreType.DMA,
            pltpu.SemaphoreType.DMA,
            pltpu.SemaphoreType.DMA,
            pltpu.SemaphoreType.DMA,
        ],
    )
    def gather_kernel(table_hbm, idx_hbm, out_hbm, spm, ib, rb0, rb1,
                      g0, g1, w0, w1):
        c = lax.axis_index("c")
        s = lax.axis_index("s")
        @pl.when(s < 10)
        def _():
            pltpu.sync_copy(table_hbm.at[pl.ds(c * N + s * 1000, 1000)],
                            spm.at[pl.ds(s * 1000, 1000)])

        plsc.subcore_barrier()

        wbase = c * half + s * per_w
        off = c * N

        @pl.loop(0, n_big)
        def _(b):
            bbase = wbase + b * _IDXB
            pltpu.sync_copy(idx_hbm.at[pl.ds(bbase, _IDXB)], ib)

            @pl.loop(0, _IDXB // 16)
            def _(k):
                sl = pl.ds(k * 16, 16)
                ib[sl] = ib[sl] - off

            @pl.loop(0, _IDXB // (2 * _WIN))
            def _(j):
                o0 = bbase + 2 * j * _WIN
                o1 = o0 + _WIN
                i0 = pl.ds((2 * j) * _WIN, _WIN)
                i1 = pl.ds((2 * j + 1) * _WIN, _WIN)
                first = jnp.logical_and(j == 0, b == 0)

                @pl.when(jnp.logical_not(first))
                def _():
                    pltpu.make_async_copy(
                        rb0, out_hbm.at[pl.ds(o0, _WIN)], w0).wait()

                h0 = pltpu.async_copy(spm.at[ib.at[i0]], rb0, g0)

                @pl.when(jnp.logical_not(first))
                def _():
                    pltpu.make_async_copy(
                        rb1, out_hbm.at[pl.ds(o1, _WIN)], w1).wait()

                h1 = pltpu.async_copy(spm.at[ib.at[i1]], rb1, g1)
                h0.wait()
                pltpu.async_copy(rb0, out_hbm.at[pl.ds(o0, _WIN)], w0)
                h1.wait()
                pltpu.async_copy(rb1, out_hbm.at[pl.ds(o1, _WIN)], w1)

        pltpu.make_async_copy(rb0, out_hbm.at[pl.ds(wbase, _WIN)], w0).wait()
        pltpu.make_async_copy(rb1, out_hbm.at[pl.ds(wbase, _WIN)], w1).wait()

    return gather_kernel


def _edge_body(e_ref, gs_ref, gd_ref, wet_ref, w1t_ref, b1_ref, gam_ref,
               bet_ref, o_ref):
    h = lax.dot_general(e_ref[...], wet_ref[...], (((1,), (0,)), ((), ())),
                        preferred_element_type=jnp.float32)
    h = h + gs_ref[0] + gd_ref[0]
    h = h * jax.nn.sigmoid(h)
    h2 = lax.dot_general(h.astype(jnp.bfloat16), w1t_ref[...],
                         (((1,), (0,)), ((), ())),
                         preferred_element_type=jnp.float32)
    h2 = h2 + b1_ref[...]
    mu = jnp.mean(h2, axis=-1, keepdims=True)
    d = h2 - mu
    var = jnp.mean(d * d, axis=-1, keepdims=True)
    o_ref[...] = d * lax.rsqrt(var + 1e-5) * gam_ref[...] + bet_ref[...]


def _edge_body_carry(e_ref, gs_ref, gd_ref, wet_ref, w1t_ref, b1_ref,
                     gam_ref, bet_ref, carry_ref, o_ref):
    del carry_ref
    _edge_body(e_ref, gs_ref, gd_ref, wet_ref, w1t_ref, b1_ref, gam_ref,
               bet_ref, o_ref)


def _edge_pass(efeat, Gr, carry, WeT, W1T, b1, gamma, beta,
               E, EF, H, OUT, BE, nblk, blk_off):
    in_specs = [
        pl.BlockSpec((BE, EF), lambda i, o=blk_off: (o + i, 0)),
        pl.BlockSpec((1, BE, H), lambda i: (0, i, 0)),
        pl.BlockSpec((1, BE, H), lambda i: (1, i, 0)),
        pl.BlockSpec((EF, H), lambda i: (0, 0)),
        pl.BlockSpec((H, OUT), lambda i: (0, 0)),
        pl.BlockSpec((1, OUT), lambda i: (0, 0)),
        pl.BlockSpec((1, OUT), lambda i: (0, 0)),
        pl.BlockSpec((1, OUT), lambda i: (0, 0)),
    ]
    args = [efeat, Gr, Gr, WeT, W1T, b1, gamma, beta]
    body = _edge_body
    kwargs = {}
    if carry is not None:
        in_specs.append(pl.BlockSpec(memory_space=pl.ANY))
        args.append(carry)
        body = _edge_body_carry
        kwargs["input_output_aliases"] = {8: 0}
    return pl.pallas_call(
        body,
        grid=(nblk,),
        in_specs=in_specs,
        out_specs=pl.BlockSpec((BE, OUT), lambda i, o=blk_off: (o + i, 0)),
        out_shape=jax.ShapeDtypeStruct((E, OUT), jnp.float32),
        **kwargs,
    )(*args)


def kernel(efeat, src_feat, dst_feat, edge_index, W_e, W_s, W_d, b0, W1, b1,
           gamma, beta):
    E, EF = efeat.shape
    N, SD = src_feat.shape
    H = W_e.shape[0]
    OUT = W1.shape[0]

    Wsd = jnp.stack([W_s, W_d])
    bsd = jnp.stack([jnp.zeros_like(b0), b0]).reshape(2, 1, H)
    T = _node_tables(src_feat, dst_feat, Wsd, bsd, N, SD, H)
    T2 = T.reshape(2 * N, H)

    S = 4
    BE = 4000
    Es = E // S
    assert E % S == 0 and Es % BE == 0
    Es_pad = -(-Es // (_IDXB * _NS)) * (_IDXB * _NS)
    P = Es_pad - Es
    pad0 = jnp.zeros((P,), jnp.int32)
    WeT = W_e.T
    W1T = W1.T.astype(jnp.bfloat16)
    b1r = b1.reshape(1, OUT)
    gr = gamma.reshape(1, OUT)
    br = beta.reshape(1, OUT)
    sc_gather = _make_sc_gather(2 * Es_pad, H, N)

    out = None
    for s in range(S):
        sl = slice(s * Es, (s + 1) * Es)
        J = jnp.concatenate([edge_index[0, sl], pad0,
                             edge_index[1, sl] + N, pad0 + N])
        G = sc_gather(T2, J)
        Gr = G.reshape(2, Es_pad, H)
        out = _edge_pass(efeat, Gr, out, WeT, W1T, b1r, gr, br,
                         E, EF, H, OUT, BE,
                         nblk=Es // BE, blk_off=s * (Es // BE))
    return out


# speedup vs baseline: 1.0408x; 1.0408x over previous
import functools

import jax
import jax.numpy as jnp
from jax import lax
from jax.experimental import pallas as pl
from jax.experimental.pallas import tpu as pltpu
from jax.experimental.pallas import tpu_sc as plsc


def _tables_body(src_ref, dst_ref, w_ref, b_ref, out_ref):
    pid = pl.program_id(0)
    x = jnp.where(pid == 0, src_ref[...], dst_ref[...])
    w = w_ref[0]
    y = lax.dot_general(x, w, (((1,), (1,)), ((), ())),
                        preferred_element_type=jnp.float32)
    out_ref[0] = y + b_ref[0]


def _node_tables(src_feat, dst_feat, Wsd, bsd, N, SD, H):
    return pl.pallas_call(
        _tables_body,
        grid=(2,),
        in_specs=[
            pl.BlockSpec((N, SD), lambda i: (0, 0)),
            pl.BlockSpec((N, SD), lambda i: (0, 0)),
            pl.BlockSpec((1, H, SD), lambda i: (i, 0, 0)),
            pl.BlockSpec((1, 1, H), lambda i: (i, 0, 0)),
        ],
        out_specs=pl.BlockSpec((1, N, H), lambda i: (i, 0, 0)),
        out_shape=jax.ShapeDtypeStruct((2, N, H), jnp.float32),
    )(src_feat, dst_feat, Wsd, bsd)


_NC = 2
_NS = 16
_NW = _NC * _NS

_WIN = 128
_IDXB = 2048


def _make_sc_gather(total, H, N):
    per_w = total // _NW
    n_big = per_w // _IDXB
    assert total % _NW == 0 and per_w % _IDXB == 0 and N % (8 * 10) == 0
    half = total // 2
    mesh = plsc.VectorSubcoreMesh(core_axis_name="c", subcore_axis_name="s")

    @functools.partial(
        pl.kernel,
        mesh=mesh,
        out_type=jax.ShapeDtypeStruct((total, H), jnp.float32),
        scratch_types=[
            pltpu.VMEM_SHARED((N, H), jnp.float32),
            pltpu.VMEM((_IDXB,), jnp.int32),
            pltpu.VMEM((_WIN, H), jnp.float32),
            pltpu.VMEM((_WIN, H), jnp.float32),
            pltpu.SemaphoreType.DMA,
            pltpu.SemaphoreType.DMA,
            pltpu.SemaphoreType.DMA,
            pltpu.SemaphoreType.DMA,
        ],
    )
    def gather_kernel(table_hbm, idx_hbm, out_hbm, spm, ib, rb0, rb1,
                      g0, g1, w0, w1):
        c = lax.axis_index("c")
        s = lax.axis_index("s")
        @pl.when(s < 10)
        def _():
            pltpu.sync_copy(table_hbm.at[pl.ds(c * N + s * 1000, 1000)],
                            spm.at[pl.ds(s * 1000, 1000)])

        plsc.subcore_barrier()

        wbase = c * half + s * per_w
        off = c * N

        @pl.loop(0, n_big)
        def _(b):
            bbase = wbase + b * _IDXB
            pltpu.sync_copy(idx_hbm.at[pl.ds(bbase, _IDXB)], ib)

            @pl.loop(0, _IDXB // 16)
            def _(k):
                sl = pl.ds(k * 16, 16)
                ib[sl] = ib[sl] - off

            @pl.loop(0, _IDXB // (2 * _WIN))
            def _(j):
                o0 = bbase + 2 * j * _WIN
                o1 = o0 + _WIN
                i0 = pl.ds((2 * j) * _WIN, _WIN)
                i1 = pl.ds((2 * j + 1) * _WIN, _WIN)
                first = jnp.logical_and(j == 0, b == 0)

                @pl.when(jnp.logical_not(first))
                def _():
                    pltpu.make_async_copy(
                        rb0, out_hbm.at[pl.ds(o0, _WIN)], w0).wait()

                h0 = pltpu.async_copy(spm.at[ib.at[i0]], rb0, g0)

                @pl.when(jnp.logical_not(first))
                def _():
                    pltpu.make_async_copy(
                        rb1, out_hbm.at[pl.ds(o1, _WIN)], w1).wait()

                h1 = pltpu.async_copy(spm.at[ib.at[i1]], rb1, g1)
                h0.wait()
                pltpu.async_copy(rb0, out_hbm.at[pl.ds(o0, _WIN)], w0)
                h1.wait()
                pltpu.async_copy(rb1, out_hbm.at[pl.ds(o1, _WIN)], w1)

        pltpu.make_async_copy(rb0, out_hbm.at[pl.ds(wbase, _WIN)], w0).wait()
        pltpu.make_async_copy(rb1, out_hbm.at[pl.ds(wbase, _WIN)], w1).wait()

    return gather_kernel


def _edge_body(e_ref, gs_ref, gd_ref, wet_ref, w1t_ref, b1_ref, gam_ref,
               bet_ref, o_ref):
    h = lax.dot_general(e_ref[...], wet_ref[...], (((1,), (0,)), ((), ())),
                        preferred_element_type=jnp.float32)
    h = h + gs_ref[0] + gd_ref[0]
    h = h * jax.nn.sigmoid(h)
    h2 = lax.dot_general(h.astype(jnp.bfloat16), w1t_ref[...],
                         (((1,), (0,)), ((), ())),
                         preferred_element_type=jnp.float32)
    h2 = h2 + b1_ref[...]
    mu = jnp.mean(h2, axis=-1, keepdims=True)
    d = h2 - mu
    var = jnp.mean(d * d, axis=-1, keepdims=True)
    o_ref[...] = d * lax.rsqrt(var + 1e-5) * gam_ref[...] + bet_ref[...]


def _edge_body_carry(e_ref, gs_ref, gd_ref, wet_ref, w1t_ref, b1_ref,
                     gam_ref, bet_ref, carry_ref, o_ref):
    del carry_ref
    _edge_body(e_ref, gs_ref, gd_ref, wet_ref, w1t_ref, b1_ref, gam_ref,
               bet_ref, o_ref)


def _edge_pass(efeat, Gr, carry, WeT, W1T, b1, gamma, beta,
               E, EF, H, OUT, BE, nblk, blk_off):
    in_specs = [
        pl.BlockSpec((BE, EF), lambda i, o=blk_off: (o + i, 0)),
        pl.BlockSpec((1, BE, H), lambda i: (0, i, 0)),
        pl.BlockSpec((1, BE, H), lambda i: (1, i, 0)),
        pl.BlockSpec((EF, H), lambda i: (0, 0)),
        pl.BlockSpec((H, OUT), lambda i: (0, 0)),
        pl.BlockSpec((1, OUT), lambda i: (0, 0)),
        pl.BlockSpec((1, OUT), lambda i: (0, 0)),
        pl.BlockSpec((1, OUT), lambda i: (0, 0)),
    ]
    args = [efeat, Gr, Gr, WeT, W1T, b1, gamma, beta]
    body = _edge_body
    kwargs = {}
    if carry is not None:
        in_specs.append(pl.BlockSpec(memory_space=pl.ANY))
        args.append(carry)
        body = _edge_body_carry
        kwargs["input_output_aliases"] = {8: 0}
    return pl.pallas_call(
        body,
        grid=(nblk,),
        in_specs=in_specs,
        out_specs=pl.BlockSpec((BE, OUT), lambda i, o=blk_off: (o + i, 0)),
        out_shape=jax.ShapeDtypeStruct((E, OUT), jnp.float32),
        **kwargs,
    )(*args)


def kernel(efeat, src_feat, dst_feat, edge_index, W_e, W_s, W_d, b0, W1, b1,
           gamma, beta):
    E, EF = efeat.shape
    N, SD = src_feat.shape
    H = W_e.shape[0]
    OUT = W1.shape[0]

    Wsd = jnp.stack([W_s, W_d])
    bsd = jnp.stack([jnp.zeros_like(b0), b0]).reshape(2, 1, H)
    T = _node_tables(src_feat, dst_feat, Wsd, bsd, N, SD, H)
    T2 = T.reshape(2 * N, H)

    S = 2
    BE = 8000
    Es = E // S
    assert E % S == 0 and Es % BE == 0
    Es_pad = -(-Es // (_IDXB * _NS)) * (_IDXB * _NS)
    P = Es_pad - Es
    pad0 = jnp.zeros((P,), jnp.int32)
    WeT = W_e.T
    W1T = W1.T.astype(jnp.bfloat16)
    b1r = b1.reshape(1, OUT)
    gr = gamma.reshape(1, OUT)
    br = beta.reshape(1, OUT)
    sc_gather = _make_sc_gather(2 * Es_pad, H, N)

    out = None
    for s in range(S):
        sl = slice(s * Es, (s + 1) * Es)
        J = jnp.concatenate([edge_index[0, sl], pad0,
                             edge_index[1, sl] + N, pad0 + N])
        G = sc_gather(T2, J)
        Gr = G.reshape(2, Es_pad, H)
        out = _edge_pass(efeat, Gr, out, WeT, W1T, b1r, gr, br,
                         E, EF, H, OUT, BE,
                         nblk=Es // BE, blk_off=s * (Es // BE))
    return out
